# Initial kernel scaffold; baseline (speedup 1.0000x reference)
import jax
import jax.numpy as jnp
from jax.experimental import pallas as pl


def kernel(feat, shuf_feat, edge_index_1, edge_index_2, edge_weight_1, edge_weight_2, W1, b1, a1, W2, b2, a2, Wb, bb):
    raise NotImplementedError("write your pallas kernel here")


# speedup vs baseline: 2.7119x; 2.7119x over previous
import functools

import jax
import jax.numpy as jnp
from jax import lax
from jax.experimental import pallas as pl
from jax.experimental.pallas import tpu as pltpu
from jax.experimental.pallas import tpu_sc as plsc

N_NODES = 10000
N_EDGES = 320000
IN_DIM = 128
OUT_DIM = 512

NP = 10240
SB = 80
NSUB = 5
CH = SB * NSUB
HD = IN_DIM // 2
EPT = N_EDGES // 16
NCHUNK = EPT // CH
NPR = 10240
ROWS_PT = NPR // 16
BINS_PT = NP // 16


def _sc_body(src_r, dst_r, w_r, src2_r, dst2_r, w2_r, tab_a, tab_b,
             agg_a, agg_b, deg_out,
             rowsv, srcv, dstv, wv, nsv, histo, histi, redv, accv, zbuf,
             agg_sh, parts_o, parts_i, ns_sh):
  t = lax.axis_index("s")
  cid = lax.axis_index("c")
  zeros16 = jnp.zeros((16,), jnp.float32)
  ones16 = jnp.ones((16,), jnp.float32)

  def _zb(r, c):
    for k in range(HD // 16):
      zbuf[r, pl.ds(k * 16, 16)] = zeros16
    return c
  lax.fori_loop(0, 128, _zb, 0)

  def _hz(g, c):
    histo[pl.ds(g * 16, 16)] = zeros16
    histi[pl.ds(g * 16, 16)] = zeros16
    return c
  lax.fori_loop(0, NP // 16, _hz, 0)

  for z in range(5):
    pltpu.sync_copy(zbuf, agg_sh.at[pl.ds(t * ROWS_PT + z * 128, 128)])

  def _hist_chunk(c, carry):
    pltpu.sync_copy(src_r.at[t, c], srcv)
    pltpu.sync_copy(dst_r.at[t, c], dstv)
    for j in range(NSUB):
      def _hb(g, cc):
        sl = pl.ds(g * 16, 16)
        plsc.addupdate_scatter(histo, [srcv[j, sl]], ones16)
        plsc.addupdate_scatter(histi, [dstv[j, sl]], ones16)
        return cc
      lax.fori_loop(0, SB // 16, _hb, 0)
    return carry
  lax.fori_loop(0, NCHUNK, _hist_chunk, 0)

  pltpu.sync_copy(histo, parts_o.at[pl.ds(t * NP, NP)])
  pltpu.sync_copy(histi, parts_i.at[pl.ds(t * NP, NP)])
  plsc.subcore_barrier()

  def _za(g, c):
    accv[pl.ds(g * 16, 16)] = zeros16
    return c

  def _radd(s, c):
    pltpu.sync_copy(parts_o.at[pl.ds(s * NP + t * BINS_PT, BINS_PT)], redv)
    def _ra(g, cc):
      sl = pl.ds(g * 16, 16)
      accv[sl] = accv[sl] + redv[sl]
      return cc
    lax.fori_loop(0, BINS_PT // 16, _ra, 0)
    return c

  lax.fori_loop(0, BINS_PT // 16, _za, 0)
  lax.fori_loop(0, 16, _radd, 0)

  def _newton(g, c):
    sl = pl.ds(g * 16, 16)
    d = jnp.maximum(accv[sl], 1.0)
    i = plsc.bitcast(d, jnp.int32)
    i = jnp.int32(0x5F3759DF) - lax.shift_right_logical(i, 1)
    y = plsc.bitcast(i, jnp.float32)
    for _ in range(3):
      y = y * (1.5 - 0.5 * d * y * y)
    redv[sl] = y
    return c
  lax.fori_loop(0, BINS_PT // 16, _newton, 0)
  pltpu.sync_copy(redv, ns_sh.at[pl.ds(t * BINS_PT, BINS_PT)])

  def _radd_in(s, c):
    pltpu.sync_copy(parts_i.at[pl.ds(s * NP + t * BINS_PT, BINS_PT)], redv)
    def _ra(g, cc):
      sl = pl.ds(g * 16, 16)
      accv[sl] = accv[sl] + redv[sl]
      return cc
    lax.fori_loop(0, BINS_PT // 16, _ra, 0)
    return c
  lax.fori_loop(0, BINS_PT // 16, _za, 0)
  lax.fori_loop(0, 16, _radd_in, 0)

  @pl.when(cid == 0)
  def _():
    pltpu.sync_copy(accv, deg_out.at[pl.ds(t * BINS_PT, BINS_PT)])

  plsc.subcore_barrier()
  pltpu.sync_copy(ns_sh, nsv)

  off = cid * N_NODES

  def _zero_agg():
    for z in range(5):
      pltpu.sync_copy(zbuf, agg_sh.at[pl.ds(t * ROWS_PT + z * 128, 128)])

  def _edge_pass(p, s_r, d_r, ww_r, fold_ns, tb_r, out_r):
    def _chunk(c, carry):
      pltpu.sync_copy(s_r.at[t, c], srcv)
      pltpu.sync_copy(d_r.at[t, c], dstv)
      pltpu.sync_copy(ww_r.at[t, c], wv)
      if fold_ns:
        for j in range(NSUB):
          def _fold(g, cc):
            sl = pl.ds(g * 16, 16)
            nsg = plsc.load_gather(nsv, [srcv[j, sl]])
            wv[j, sl] = wv[j, sl] * nsg
            return cc
          lax.fori_loop(0, SB // 16, _fold, 0)
      for j in range(NSUB):
        def _off(g, cc):
          sl = pl.ds(g * 16, 16)
          srcv[j, sl] = srcv[j, sl] + off
          return cc
        lax.fori_loop(0, SB // 16, _off, 0)
      for j in range(NSUB):
        pltpu.sync_copy(tb_r.at[srcv.at[j]], rowsv.at[pl.ds(j * SB, SB)])
      for j in range(NSUB):
        def _scale(g, cc):
          wg = wv[j, pl.ds(g * 16, 16)]
          for lane in range(16):
            wsc = wg[lane]
            e = j * SB + g * 16 + lane
            for k in range(HD // 16):
              sl = pl.ds(k * 16, 16)
              rowsv[e, sl] = rowsv[e, sl] * wsc
          return cc
        lax.fori_loop(0, SB // 16, _scale, 0)
      for j in range(NSUB):
        pltpu.sync_copy(rowsv.at[pl.ds(j * SB, SB)], agg_sh.at[dstv.at[j]],
                        add=True)
      return carry
    lax.fori_loop(0, NCHUNK, _chunk, 0)
    plsc.subcore_barrier()
    row_base = (cid * 2 + p) * NPR + t * ROWS_PT
    pltpu.sync_copy(agg_sh.at[pl.ds(t * ROWS_PT, ROWS_PT)],
                    out_r.at[pl.ds(row_base, ROWS_PT)])

  passes = [
      (0, src_r, dst_r, w_r, True, tab_a, agg_a),
      (0, src_r, dst_r, w_r, True, tab_b, agg_b),
      (1, src2_r, dst2_r, w2_r, False, tab_a, agg_a),
      (1, src2_r, dst2_r, w2_r, False, tab_b, agg_b),
  ]
  for pi, args in enumerate(passes):
    _edge_pass(*args)
    if pi < 3:
      _zero_agg()
      plsc.subcore_barrier()


@jax.jit
def _sc_aggregate(src1, dst1, w1, src2, dst2, w2, tab_a, tab_b):
  mesh = plsc.VectorSubcoreMesh(core_axis_name="c", subcore_axis_name="s",
                                num_cores=2, num_subcores=16)
  f32 = jnp.float32
  kern = pl.kernel(
      _sc_body,
      out_type=[
          jax.ShapeDtypeStruct((4 * NPR, HD), f32),
          jax.ShapeDtypeStruct((4 * NPR, HD), f32),
          jax.ShapeDtypeStruct((NP,), f32),
      ],
      mesh=mesh,
      compiler_params=pltpu.CompilerParams(needs_layout_passes=False,
                                           use_tc_tiling_on_sc=False),
      scratch_types=[
          pltpu.VMEM((CH, HD), f32),
          pltpu.VMEM((NSUB, SB), jnp.int32),
          pltpu.VMEM((NSUB, SB), jnp.int32),
          pltpu.VMEM((NSUB, SB), f32),
          pltpu.VMEM((NP,), f32),
          pltpu.VMEM((NP,), f32),
          pltpu.VMEM((NP,), f32),
          pltpu.VMEM((BINS_PT,), f32),
          pltpu.VMEM((BINS_PT,), f32),
          pltpu.VMEM((128, HD), f32),
          pltpu.VMEM_SHARED((NPR, HD), f32),
          pltpu.VMEM_SHARED((16 * NP,), f32),
          pltpu.VMEM_SHARED((16 * NP,), f32),
          pltpu.VMEM_SHARED((NP,), f32),
      ],
  )
  return kern(src1, dst1, w1, src2, dst2, w2, tab_a, tab_b)


_R = 1000
_G = N_NODES // _R


def _prelu(z, a):
  return jnp.where(z >= 0, z, a * z)


def _means_body(a1_r, a2_r, deg_r, W1_r, b1_r, aa1_r, W2_r, b2_r, aa2_r,
                s1_r, s2_r):
  i = pl.program_id(0)
  nd = lax.rsqrt(jnp.maximum(deg_r[...], 1.0))
  z1 = jnp.dot(a1_r[...] * nd, W1_r[...],
               preferred_element_type=jnp.float32) + b1_r[...]
  h1 = _prelu(z1, aa1_r[0, 0])
  z2 = jnp.dot(a2_r[...], W2_r[...],
               preferred_element_type=jnp.float32) + b2_r[...]
  h2 = _prelu(z2, aa2_r[0, 0])

  @pl.when(i == 0)
  def _():
    s1_r[...] = jnp.zeros_like(s1_r)
    s2_r[...] = jnp.zeros_like(s2_r)

  s1_r[...] += jnp.sum(h1, axis=0, keepdims=True)
  s2_r[...] += jnp.sum(h2, axis=0, keepdims=True)


@jax.jit
def _tc_means(agg1, agg2, deg, W1, b1, a1, W2, b2, a2):
  f32 = jnp.float32
  blk = lambda r, c: pl.BlockSpec((r, c), lambda i: (i, 0))
  full = lambda r, c: pl.BlockSpec((r, c), lambda i: (0, 0))
  return pl.pallas_call(
      _means_body,
      grid=(_G,),
      in_specs=[blk(_R, IN_DIM), blk(_R, IN_DIM), blk(_R, 1),
                full(IN_DIM, OUT_DIM), full(1, OUT_DIM), full(1, 1),
                full(IN_DIM, OUT_DIM), full(1, OUT_DIM), full(1, 1)],
      out_specs=[full(1, OUT_DIM), full(1, OUT_DIM)],
      out_shape=[jax.ShapeDtypeStruct((1, OUT_DIM), f32),
                 jax.ShapeDtypeStruct((1, OUT_DIM), f32)],
  )(agg1, agg2, deg, W1, b1, a1, W2, b2, a2)


def _bilin_body(Wb_r, c_r, v_r):
  v_r[...] = jnp.dot(Wb_r[...], c_r[...], preferred_element_type=jnp.float32)


@jax.jit
def _tc_bilin_vec(Wb, c12):
  return pl.pallas_call(
      _bilin_body,
      out_shape=jax.ShapeDtypeStruct((OUT_DIM, 2), jnp.float32),
  )(Wb, c12)


def _scores_body(a1_r, a2_r, a3_r, a4_r, deg_r, W1_r, b1_r, aa1_r,
                 W2_r, b2_r, aa2_r, v_r, bb_r,
                 o1_r, o2_r, o3_r, o4_r):
  nd = lax.rsqrt(jnp.maximum(deg_r[...], 1.0))
  dotf = lambda x, w: jnp.dot(x, w, preferred_element_type=jnp.float32)
  h1 = _prelu(dotf(a1_r[...] * nd, W1_r[...]) + b1_r[...], aa1_r[0, 0])
  h2 = _prelu(dotf(a2_r[...], W2_r[...]) + b2_r[...], aa2_r[0, 0])
  h3 = _prelu(dotf(a3_r[...] * nd, W1_r[...]) + b1_r[...], aa1_r[0, 0])
  h4 = _prelu(dotf(a4_r[...], W2_r[...]) + b2_r[...], aa2_r[0, 0])
  v1 = v_r[:, 0:1]
  v2 = v_r[:, 1:2]
  bb = bb_r[0, 0]
  o1_r[...] = dotf(h2, v1) + bb
  o2_r[...] = dotf(h1, v2) + bb
  o3_r[...] = dotf(h4, v1) + bb
  o4_r[...] = dotf(h3, v2) + bb


@jax.jit
def _tc_scores(agg1, agg2, agg3, agg4, deg, W1, b1, a1, W2, b2, a2, v12, bb):
  f32 = jnp.float32
  blk = lambda r, c: pl.BlockSpec((r, c), lambda i: (i, 0))
  full = lambda r, c: pl.BlockSpec((r, c), lambda i: (0, 0))
  outs = pl.pallas_call(
      _scores_body,
      grid=(_G,),
      in_specs=[blk(_R, IN_DIM)] * 4 + [blk(_R, 1),
                full(IN_DIM, OUT_DIM), full(1, OUT_DIM), full(1, 1),
                full(IN_DIM, OUT_DIM), full(1, OUT_DIM), full(1, 1),
                full(OUT_DIM, 2), full(1, 1)],
      out_specs=[blk(_R, 1)] * 4,
      out_shape=[jax.ShapeDtypeStruct((N_NODES, 1), f32)] * 4,
  )(agg1, agg2, agg3, agg4, deg, W1, b1, a1, W2, b2, a2, v12, bb)
  return outs


def kernel(feat, shuf_feat, edge_index_1, edge_index_2, edge_weight_1,
           edge_weight_2, W1, b1, a1, W2, b2, a2, Wb, bb):
  f32 = jnp.float32
  esh = (16, NCHUNK, NSUB, SB)
  src1 = edge_index_1[0].astype(jnp.int32).reshape(esh)
  dst1 = edge_index_1[1].astype(jnp.int32).reshape(esh)
  src2 = edge_index_2[0].astype(jnp.int32).reshape(esh)
  dst2 = edge_index_2[1].astype(jnp.int32).reshape(esh)
  w1 = edge_weight_1.reshape(esh)
  w2 = edge_weight_2.reshape(esh)
  tab = jnp.concatenate([feat, shuf_feat], axis=0)
  tab_a = tab[:, :HD]
  tab_b = tab[:, HD:]

  agg_ha, agg_hb, deg = _sc_aggregate(src1, dst1, w1, src2, dst2, w2,
                                      tab_a, tab_b)
  aggs = jnp.concatenate([agg_ha, agg_hb], axis=1)
  agg1 = aggs[0 * NPR:0 * NPR + N_NODES]
  agg2 = aggs[1 * NPR:1 * NPR + N_NODES]
  agg3 = aggs[2 * NPR:2 * NPR + N_NODES]
  agg4 = aggs[3 * NPR:3 * NPR + N_NODES]
  degc = deg[:N_NODES].reshape(N_NODES, 1)

  b1r = b1.reshape(1, OUT_DIM)
  b2r = b2.reshape(1, OUT_DIM)
  a1r = a1.reshape(1, 1)
  a2r = a2.reshape(1, 1)
  bbr = bb.reshape(1, 1)

  s1, s2 = _tc_means(agg1, agg2, degc, W1, b1r, a1r, W2, b2r, a2r)
  c1 = jax.nn.sigmoid(s1[0] / N_NODES)
  c2 = jax.nn.sigmoid(s2[0] / N_NODES)
  c12 = jnp.stack([c1, c2], axis=1)
  v12 = _tc_bilin_vec(Wb, c12)

  o1, o2, o3, o4 = _tc_scores(agg1, agg2, agg3, agg4, degc,
                              W1, b1r, a1r, W2, b2r, a2r, v12, bbr)
  return jnp.concatenate([o1[:, 0], o2[:, 0], o3[:, 0], o4[:, 0]])


# speedup vs baseline: 4.1975x; 1.5478x over previous
import functools

import jax
import jax.numpy as jnp
from jax import lax
from jax.experimental import pallas as pl
from jax.experimental.pallas import tpu as pltpu
from jax.experimental.pallas import tpu_sc as plsc

N_NODES = 10000
N_EDGES = 320000
IN_DIM = 128
OUT_DIM = 512

NP = 10240
SB = 80
NSUB = 2
CH = SB * NSUB
HD = IN_DIM // 2
EPT = N_EDGES // 16
NCHUNK = EPT // CH
NPR = 10240
ROWS_PT = NPR // 16
BINS_PT = NP // 16


def _sc_body(src_r, dst_r, w_r, src2_r, dst2_r, w2_r, tab_a, tab_b,
             agg_a, agg_b, deg_out,
             rows_all, srcv_all, dstv_all, wv_all, nsv, histo, histi,
             redv, accv, zbuf,
             agg_sh, parts_o, parts_i, ns_sh,
             gsems, ssems, isems):
  t = lax.axis_index("s")
  cid = lax.axis_index("c")
  zeros16 = jnp.zeros((16,), jnp.float32)
  ones16 = jnp.ones((16,), jnp.float32)

  def _zb(r, c):
    for k in range(HD // 16):
      zbuf[r, pl.ds(k * 16, 16)] = zeros16
    return c
  lax.fori_loop(0, zbuf.shape[0], _zb, 0)

  def _hz(g, c):
    histo[pl.ds(g * 16, 16)] = zeros16
    histi[pl.ds(g * 16, 16)] = zeros16
    return c
  lax.fori_loop(0, NP // 16, _hz, 0)

  def _zero_agg():
    nz = ROWS_PT // zbuf.shape[0]
    for z in range(nz):
      pltpu.sync_copy(
          zbuf, agg_sh.at[pl.ds(t * ROWS_PT + z * zbuf.shape[0],
                                zbuf.shape[0])])
  _zero_agg()

  def _hload_start(c, b):
    pltpu.async_copy(src_r.at[t, c], srcv_all.at[b], isems.at[b])
    pltpu.async_copy(dst_r.at[t, c], dstv_all.at[b], isems.at[b])

  def _hload_wait(c, b):
    pltpu.make_async_copy(src_r.at[t, c], srcv_all.at[b], isems.at[b]).wait()
    pltpu.make_async_copy(dst_r.at[t, c], dstv_all.at[b], isems.at[b]).wait()

  _hload_start(0, 0)

  def _hist_chunk(c, carry):
    b = lax.rem(c, 3)
    bn = lax.rem(c + 1, 3)

    @pl.when(c + 1 < NCHUNK)
    def _():
      _hload_start(c + 1, bn)

    _hload_wait(c, b)
    for j in range(NSUB):
      def _hb(g, cc):
        sl = pl.ds(g * 16, 16)
        plsc.addupdate_scatter(histo, [srcv_all[b, j, sl]], ones16)
        plsc.addupdate_scatter(histi, [dstv_all[b, j, sl]], ones16)
        return cc
      lax.fori_loop(0, SB // 16, _hb, 0)
    return carry
  lax.fori_loop(0, NCHUNK, _hist_chunk, 0)

  pltpu.sync_copy(histo, parts_o.at[pl.ds(t * NP, NP)])
  pltpu.sync_copy(histi, parts_i.at[pl.ds(t * NP, NP)])
  plsc.subcore_barrier()

  def _za(g, c):
    accv[pl.ds(g * 16, 16)] = zeros16
    return c

  def _pload(parts, buf):
    for s in range(16):
      pltpu.async_copy(parts.at[pl.ds(s * NP + t * BINS_PT, BINS_PT)],
                       buf.at[pl.ds(s * BINS_PT, BINS_PT)], isems.at[0])
    for s in range(16):
      pltpu.make_async_copy(
          parts.at[pl.ds(s * NP + t * BINS_PT, BINS_PT)],
          buf.at[pl.ds(s * BINS_PT, BINS_PT)], isems.at[0]).wait()

  def _reduce(buf):
    def _rg(g, c):
      sl = pl.ds(g * 16, 16)
      def _rs(s, acc):
        return acc + buf[pl.ds(s * BINS_PT + g * 16, 16)]
      accv[sl] = lax.fori_loop(0, 16, _rs, zeros16)
      return c
    lax.fori_loop(0, BINS_PT // 16, _rg, 0)

  _pload(parts_o, histo)
  _reduce(histo)

  def _newton(g, c):
    sl = pl.ds(g * 16, 16)
    d = jnp.maximum(accv[sl], 1.0)
    i = plsc.bitcast(d, jnp.int32)
    i = jnp.int32(0x5F3759DF) - lax.shift_right_logical(i, 1)
    y = plsc.bitcast(i, jnp.float32)
    for _ in range(3):
      y = y * (1.5 - 0.5 * d * y * y)
    redv[sl] = y
    return c
  lax.fori_loop(0, BINS_PT // 16, _newton, 0)
  pltpu.sync_copy(redv, ns_sh.at[pl.ds(t * BINS_PT, BINS_PT)])

  _pload(parts_i, histi)
  _reduce(histi)

  @pl.when(cid == 0)
  def _():
    pltpu.sync_copy(accv, deg_out.at[pl.ds(t * BINS_PT, BINS_PT)])

  plsc.subcore_barrier()
  pltpu.sync_copy(ns_sh, nsv)

  off = cid * N_NODES

  def _edge_pass(p, s_r, d_r, ww_r, fold_ns, tb_r, out_r):
    def iload_start(c, b):
      pltpu.async_copy(s_r.at[t, c], srcv_all.at[b], isems.at[b])
      pltpu.async_copy(d_r.at[t, c], dstv_all.at[b], isems.at[b])
      pltpu.async_copy(ww_r.at[t, c], wv_all.at[b], isems.at[b])

    def iload_wait(c, b):
      pltpu.make_async_copy(s_r.at[t, c], srcv_all.at[b], isems.at[b]).wait()
      pltpu.make_async_copy(d_r.at[t, c], dstv_all.at[b], isems.at[b]).wait()
      pltpu.make_async_copy(ww_r.at[t, c], wv_all.at[b], isems.at[b]).wait()

    def fold_offset(b):
      for j in range(NSUB):
        def _fo(g, cc):
          sl = pl.ds(g * 16, 16)
          if fold_ns:
            nsg = plsc.load_gather(nsv, [srcv_all[b, j, sl]])
            wv_all[b, j, sl] = wv_all[b, j, sl] * nsg
          srcv_all[b, j, sl] = srcv_all[b, j, sl] + off
          return cc
        lax.fori_loop(0, SB // 16, _fo, 0)

    def gather_start(b):
      for j in range(NSUB):
        pltpu.async_copy(tb_r.at[srcv_all.at[b, j]],
                         rows_all.at[b, pl.ds(j * SB, SB)], gsems.at[b])

    def gather_wait(b):
      for j in range(NSUB):
        pltpu.make_async_copy(tb_r.at[srcv_all.at[b, j]],
                              rows_all.at[b, pl.ds(j * SB, SB)],
                              gsems.at[b]).wait()

    def scale(b):
      for j in range(NSUB):
        def _sc(g, cc):
          wg = wv_all[b, j, pl.ds(g * 16, 16)]
          for lane in range(16):
            wsc = wg[lane]
            e = j * SB + g * 16 + lane
            for k in range(HD // 16):
              sl = pl.ds(k * 16, 16)
              rows_all[b, e, sl] = rows_all[b, e, sl] * wsc
          return cc
        lax.fori_loop(0, SB // 16, _sc, 0)

    def scatter_start(b):
      for j in range(NSUB):
        pltpu.async_copy(rows_all.at[b, pl.ds(j * SB, SB)],
                         agg_sh.at[dstv_all.at[b, j]], ssems.at[b],
                         add=True)

    def scatter_wait(b):
      for j in range(NSUB):
        pltpu.make_async_copy(rows_all.at[b, pl.ds(j * SB, SB)],
                              agg_sh.at[dstv_all.at[b, j]],
                              ssems.at[b]).wait()

    iload_start(0, 0)
    iload_wait(0, 0)
    fold_offset(0)
    gather_start(0)

    def _chunk(c, carry):
      b = lax.rem(c, 3)
      bn = lax.rem(c + 1, 3)

      @pl.when(c >= 2)
      def _():
        scatter_wait(bn)

      @pl.when(c + 1 < NCHUNK)
      def _():
        iload_start(c + 1, bn)

      gather_wait(b)

      @pl.when(c + 1 < NCHUNK)
      def _():
        iload_wait(c + 1, bn)
        fold_offset(bn)
        gather_start(bn)

      scale(b)
      scatter_start(b)
      return carry
    lax.fori_loop(0, NCHUNK, _chunk, 0)

    scatter_wait((NCHUNK - 2) % 3)
    scatter_wait((NCHUNK - 1) % 3)
    plsc.subcore_barrier()
    row_base = (cid * 2 + p) * NPR + t * ROWS_PT
    pltpu.sync_copy(agg_sh.at[pl.ds(t * ROWS_PT, ROWS_PT)],
                    out_r.at[pl.ds(row_base, ROWS_PT)])

  passes = [
      (0, src_r, dst_r, w_r, True, tab_a, agg_a),
      (0, src_r, dst_r, w_r, True, tab_b, agg_b),
      (1, src2_r, dst2_r, w2_r, False, tab_a, agg_a),
      (1, src2_r, dst2_r, w2_r, False, tab_b, agg_b),
  ]
  for pi, args in enumerate(passes):
    _edge_pass(*args)
    if pi < 3:
      _zero_agg()
      plsc.subcore_barrier()


@jax.jit
def _sc_aggregate(src1, dst1, w1, src2, dst2, w2, tab_a, tab_b):
  mesh = plsc.VectorSubcoreMesh(core_axis_name="c", subcore_axis_name="s",
                                num_cores=2, num_subcores=16)
  f32 = jnp.float32
  kern = pl.kernel(
      _sc_body,
      out_type=[
          jax.ShapeDtypeStruct((4 * NPR, HD), f32),
          jax.ShapeDtypeStruct((4 * NPR, HD), f32),
          jax.ShapeDtypeStruct((NP,), f32),
      ],
      mesh=mesh,
      compiler_params=pltpu.CompilerParams(needs_layout_passes=False,
                                           use_tc_tiling_on_sc=False),
      scratch_types=[
          pltpu.VMEM((3, CH, HD), f32),
          pltpu.VMEM((3, NSUB, SB), jnp.int32),
          pltpu.VMEM((3, NSUB, SB), jnp.int32),
          pltpu.VMEM((3, NSUB, SB), f32),
          pltpu.VMEM((NP,), f32),
          pltpu.VMEM((NP,), f32),
          pltpu.VMEM((NP,), f32),
          pltpu.VMEM((BINS_PT,), f32),
          pltpu.VMEM((BINS_PT,), f32),
          pltpu.VMEM((64, HD), f32),
          pltpu.VMEM_SHARED((NPR, HD), f32),
          pltpu.HBM((16 * NP,), f32),
          pltpu.HBM((16 * NP,), f32),
          pltpu.VMEM_SHARED((NP,), f32),
          pltpu.SemaphoreType.DMA((3,)),
          pltpu.SemaphoreType.DMA((3,)),
          pltpu.SemaphoreType.DMA((3,)),
      ],
  )
  return kern(src1, dst1, w1, src2, dst2, w2, tab_a, tab_b)


_R = 1000
_G = N_NODES // _R


def _prelu(z, a):
  return jnp.where(z >= 0, z, a * z)


def _means_body(a1_r, a2_r, deg_r, W1_r, b1_r, aa1_r, W2_r, b2_r, aa2_r,
                s1_r, s2_r):
  i = pl.program_id(0)
  nd = lax.rsqrt(jnp.maximum(deg_r[...], 1.0))
  z1 = jnp.dot(a1_r[...] * nd, W1_r[...],
               preferred_element_type=jnp.float32) + b1_r[...]
  h1 = _prelu(z1, aa1_r[0, 0])
  z2 = jnp.dot(a2_r[...], W2_r[...],
               preferred_element_type=jnp.float32) + b2_r[...]
  h2 = _prelu(z2, aa2_r[0, 0])

  @pl.when(i == 0)
  def _():
    s1_r[...] = jnp.zeros_like(s1_r)
    s2_r[...] = jnp.zeros_like(s2_r)

  s1_r[...] += jnp.sum(h1, axis=0, keepdims=True)
  s2_r[...] += jnp.sum(h2, axis=0, keepdims=True)


@jax.jit
def _tc_means(agg1, agg2, deg, W1, b1, a1, W2, b2, a2):
  f32 = jnp.float32
  blk = lambda r, c: pl.BlockSpec((r, c), lambda i: (i, 0))
  full = lambda r, c: pl.BlockSpec((r, c), lambda i: (0, 0))
  return pl.pallas_call(
      _means_body,
      grid=(_G,),
      in_specs=[blk(_R, IN_DIM), blk(_R, IN_DIM), blk(_R, 1),
                full(IN_DIM, OUT_DIM), full(1, OUT_DIM), full(1, 1),
                full(IN_DIM, OUT_DIM), full(1, OUT_DIM), full(1, 1)],
      out_specs=[full(1, OUT_DIM), full(1, OUT_DIM)],
      out_shape=[jax.ShapeDtypeStruct((1, OUT_DIM), f32),
                 jax.ShapeDtypeStruct((1, OUT_DIM), f32)],
  )(agg1, agg2, deg, W1, b1, a1, W2, b2, a2)


def _bilin_body(Wb_r, c_r, v_r):
  v_r[...] = jnp.dot(Wb_r[...], c_r[...], preferred_element_type=jnp.float32)


@jax.jit
def _tc_bilin_vec(Wb, c12):
  return pl.pallas_call(
      _bilin_body,
      out_shape=jax.ShapeDtypeStruct((OUT_DIM, 2), jnp.float32),
  )(Wb, c12)


def _scores_body(a1_r, a2_r, a3_r, a4_r, deg_r, W1_r, b1_r, aa1_r,
                 W2_r, b2_r, aa2_r, v_r, bb_r,
                 o1_r, o2_r, o3_r, o4_r):
  nd = lax.rsqrt(jnp.maximum(deg_r[...], 1.0))
  dotf = lambda x, w: jnp.dot(x, w, preferred_element_type=jnp.float32)
  h1 = _prelu(dotf(a1_r[...] * nd, W1_r[...]) + b1_r[...], aa1_r[0, 0])
  h2 = _prelu(dotf(a2_r[...], W2_r[...]) + b2_r[...], aa2_r[0, 0])
  h3 = _prelu(dotf(a3_r[...] * nd, W1_r[...]) + b1_r[...], aa1_r[0, 0])
  h4 = _prelu(dotf(a4_r[...], W2_r[...]) + b2_r[...], aa2_r[0, 0])
  v1 = v_r[:, 0:1]
  v2 = v_r[:, 1:2]
  bb = bb_r[0, 0]
  o1_r[...] = dotf(h2, v1) + bb
  o2_r[...] = dotf(h1, v2) + bb
  o3_r[...] = dotf(h4, v1) + bb
  o4_r[...] = dotf(h3, v2) + bb


@jax.jit
def _tc_scores(agg1, agg2, agg3, agg4, deg, W1, b1, a1, W2, b2, a2, v12, bb):
  f32 = jnp.float32
  blk = lambda r, c: pl.BlockSpec((r, c), lambda i: (i, 0))
  full = lambda r, c: pl.BlockSpec((r, c), lambda i: (0, 0))
  outs = pl.pallas_call(
      _scores_body,
      grid=(_G,),
      in_specs=[blk(_R, IN_DIM)] * 4 + [blk(_R, 1),
                full(IN_DIM, OUT_DIM), full(1, OUT_DIM), full(1, 1),
                full(IN_DIM, OUT_DIM), full(1, OUT_DIM), full(1, 1),
                full(OUT_DIM, 2), full(1, 1)],
      out_specs=[blk(_R, 1)] * 4,
      out_shape=[jax.ShapeDtypeStruct((N_NODES, 1), f32)] * 4,
  )(agg1, agg2, agg3, agg4, deg, W1, b1, a1, W2, b2, a2, v12, bb)
  return outs


def kernel(feat, shuf_feat, edge_index_1, edge_index_2, edge_weight_1,
           edge_weight_2, W1, b1, a1, W2, b2, a2, Wb, bb):
  f32 = jnp.float32
  esh = (16, NCHUNK, NSUB, SB)
  src1 = edge_index_1[0].astype(jnp.int32).reshape(esh)
  dst1 = edge_index_1[1].astype(jnp.int32).reshape(esh)
  src2 = edge_index_2[0].astype(jnp.int32).reshape(esh)
  dst2 = edge_index_2[1].astype(jnp.int32).reshape(esh)
  w1 = edge_weight_1.reshape(esh)
  w2 = edge_weight_2.reshape(esh)
  tab = jnp.concatenate([feat, shuf_feat], axis=0)
  tab_a = tab[:, :HD]
  tab_b = tab[:, HD:]

  agg_ha, agg_hb, deg = _sc_aggregate(src1, dst1, w1, src2, dst2, w2,
                                      tab_a, tab_b)
  aggs = jnp.concatenate([agg_ha, agg_hb], axis=1)
  agg1 = aggs[0 * NPR:0 * NPR + N_NODES]
  agg2 = aggs[1 * NPR:1 * NPR + N_NODES]
  agg3 = aggs[2 * NPR:2 * NPR + N_NODES]
  agg4 = aggs[3 * NPR:3 * NPR + N_NODES]
  degc = deg[:N_NODES].reshape(N_NODES, 1)

  b1r = b1.reshape(1, OUT_DIM)
  b2r = b2.reshape(1, OUT_DIM)
  a1r = a1.reshape(1, 1)
  a2r = a2.reshape(1, 1)
  bbr = bb.reshape(1, 1)

  s1, s2 = _tc_means(agg1, agg2, degc, W1, b1r, a1r, W2, b2r, a2r)
  c1 = jax.nn.sigmoid(s1[0] / N_NODES)
  c2 = jax.nn.sigmoid(s2[0] / N_NODES)
  c12 = jnp.stack([c1, c2], axis=1)
  v12 = _tc_bilin_vec(Wb, c12)

  o1, o2, o3, o4 = _tc_scores(agg1, agg2, agg3, agg4, degc,
                              W1, b1r, a1r, W2, b2r, a2r, v12, bbr)
  return jnp.concatenate([o1[:, 0], o2[:, 0], o3[:, 0], o4[:, 0]])


# speedup vs baseline: 6.5793x; 1.5674x over previous
import functools

import jax
import jax.numpy as jnp
from jax import lax
from jax.experimental import pallas as pl
from jax.experimental.pallas import tpu as pltpu
from jax.experimental.pallas import tpu_sc as plsc

N_NODES = 10000
N_EDGES = 320000
IN_DIM = 128
OUT_DIM = 512

NP = 10240
SB = 80
NSUB = 2
CH = SB * NSUB
HD = IN_DIM // 2
EPT = N_EDGES // 16
NCHUNK = EPT // CH
NPR = 10240
ROWS_PT = NPR // 16
BINS_PT = NP // 16


def _sc_body(src_r, dst_r, w_r, src2_r, dst2_r, w2_r, tab_a, tab_b,
             agg_a, agg_b, deg_out,
             rows_all, srcv_all, dstv_all, wv_all, nsv, histo, histi,
             redv, accv, zbuf,
             agg_sh, parts_o, parts_i, ns_sh,
             gsems, ssems, isems):
  t = lax.axis_index("s")
  cid = lax.axis_index("c")
  zeros16 = jnp.zeros((16,), jnp.float32)
  ones16 = jnp.ones((16,), jnp.float32)

  def _zb(r, c):
    for k in range(HD // 16):
      zbuf[r, pl.ds(k * 16, 16)] = zeros16
    return c
  lax.fori_loop(0, zbuf.shape[0], _zb, 0)

  def _hz(g, c):
    histo[pl.ds(g * 16, 16)] = zeros16
    histi[pl.ds(g * 16, 16)] = zeros16
    return c
  lax.fori_loop(0, NP // 16, _hz, 0)

  def _zero_agg():
    nz = ROWS_PT // zbuf.shape[0]
    for z in range(nz):
      pltpu.sync_copy(
          zbuf, agg_sh.at[pl.ds(t * ROWS_PT + z * zbuf.shape[0],
                                zbuf.shape[0])])
  _zero_agg()

  def _hload_start(c, b):
    pltpu.async_copy(src_r.at[t, c], srcv_all.at[b], isems.at[b])
    pltpu.async_copy(dst_r.at[t, c], dstv_all.at[b], isems.at[b])

  def _hload_wait(c, b):
    pltpu.make_async_copy(src_r.at[t, c], srcv_all.at[b], isems.at[b]).wait()
    pltpu.make_async_copy(dst_r.at[t, c], dstv_all.at[b], isems.at[b]).wait()

  _hload_start(0, 0)

  def _hist_chunk(c, carry):
    b = lax.rem(c, 3)
    bn = lax.rem(c + 1, 3)

    @pl.when(c + 1 < NCHUNK)
    def _():
      _hload_start(c + 1, bn)

    _hload_wait(c, b)
    for j in range(NSUB):
      def _hb(g, cc):
        sl = pl.ds(g * 16, 16)
        plsc.addupdate_scatter(histo, [srcv_all[b, j, sl]], ones16)
        plsc.addupdate_scatter(histi, [dstv_all[b, j, sl]], ones16)
        return cc
      lax.fori_loop(0, SB // 16, _hb, 0)
    return carry
  lax.fori_loop(0, NCHUNK, _hist_chunk, 0)

  pltpu.sync_copy(histo, parts_o.at[pl.ds(t * NP, NP)])
  pltpu.sync_copy(histi, parts_i.at[pl.ds(t * NP, NP)])
  plsc.subcore_barrier()

  def _za(g, c):
    accv[pl.ds(g * 16, 16)] = zeros16
    return c

  def _pload(parts, buf):
    for s in range(16):
      pltpu.async_copy(parts.at[pl.ds(s * NP + t * BINS_PT, BINS_PT)],
                       buf.at[pl.ds(s * BINS_PT, BINS_PT)], isems.at[0])
    for s in range(16):
      pltpu.make_async_copy(
          parts.at[pl.ds(s * NP + t * BINS_PT, BINS_PT)],
          buf.at[pl.ds(s * BINS_PT, BINS_PT)], isems.at[0]).wait()

  def _reduce(buf):
    def _rg(g, c):
      sl = pl.ds(g * 16, 16)
      def _rs(s, acc):
        return acc + buf[pl.ds(s * BINS_PT + g * 16, 16)]
      accv[sl] = lax.fori_loop(0, 16, _rs, zeros16)
      return c
    lax.fori_loop(0, BINS_PT // 16, _rg, 0)

  _pload(parts_o, histo)
  _reduce(histo)

  def _newton(g, c):
    sl = pl.ds(g * 16, 16)
    d = jnp.maximum(accv[sl], 1.0)
    i = plsc.bitcast(d, jnp.int32)
    i = jnp.int32(0x5F3759DF) - lax.shift_right_logical(i, 1)
    y = plsc.bitcast(i, jnp.float32)
    for _ in range(3):
      y = y * (1.5 - 0.5 * d * y * y)
    redv[sl] = y
    return c
  lax.fori_loop(0, BINS_PT // 16, _newton, 0)
  pltpu.sync_copy(redv, ns_sh.at[pl.ds(t * BINS_PT, BINS_PT)])

  _pload(parts_i, histi)
  _reduce(histi)

  @pl.when(cid == 0)
  def _():
    pltpu.sync_copy(accv, deg_out.at[pl.ds(t * BINS_PT, BINS_PT)])

  plsc.subcore_barrier()
  pltpu.sync_copy(ns_sh, nsv)

  off = cid * N_NODES

  def _edge_pass(p, s_r, d_r, ww_r, fold_ns, tb_r, out_r):
    def iload_start(c, b):
      pltpu.async_copy(s_r.at[t, c], srcv_all.at[b], isems.at[b])
      pltpu.async_copy(d_r.at[t, c], dstv_all.at[b], isems.at[b])
      pltpu.async_copy(ww_r.at[t, c], wv_all.at[b], isems.at[b])

    def iload_wait(c, b):
      pltpu.make_async_copy(s_r.at[t, c], srcv_all.at[b], isems.at[b]).wait()
      pltpu.make_async_copy(d_r.at[t, c], dstv_all.at[b], isems.at[b]).wait()
      pltpu.make_async_copy(ww_r.at[t, c], wv_all.at[b], isems.at[b]).wait()

    def fold_offset(b):
      for j in range(NSUB):
        def _fo(g, cc):
          sl = pl.ds(g * 16, 16)
          if fold_ns:
            nsg = plsc.load_gather(nsv, [srcv_all[b, j, sl]])
            wv_all[b, j, sl] = wv_all[b, j, sl] * nsg
          srcv_all[b, j, sl] = srcv_all[b, j, sl] + off
          return cc
        lax.fori_loop(0, SB // 16, _fo, 0)

    def gather_start(b):
      for j in range(NSUB):
        pltpu.async_copy(tb_r.at[srcv_all.at[b, j]],
                         rows_all.at[b, pl.ds(j * SB, SB)], gsems.at[b])

    def gather_wait(b):
      for j in range(NSUB):
        pltpu.make_async_copy(tb_r.at[srcv_all.at[b, j]],
                              rows_all.at[b, pl.ds(j * SB, SB)],
                              gsems.at[b]).wait()

    def scale(b):
      for j in range(NSUB):
        @plsc.parallel_loop(0, SB // 16)
        def _sc(g):
          wg = wv_all[b, j, pl.ds(g * 16, 16)]
          for lane in range(16):
            wsc = wg[lane]
            e = j * SB + g * 16 + lane
            vals = [rows_all[b, e, pl.ds(k * 16, 16)]
                    for k in range(HD // 16)]
            for k in range(HD // 16):
              rows_all[b, e, pl.ds(k * 16, 16)] = vals[k] * wsc

    def scatter_start(b):
      for j in range(NSUB):
        pltpu.async_copy(rows_all.at[b, pl.ds(j * SB, SB)],
                         agg_sh.at[dstv_all.at[b, j]], ssems.at[b],
                         add=True)

    def scatter_wait(b):
      for j in range(NSUB):
        pltpu.make_async_copy(rows_all.at[b, pl.ds(j * SB, SB)],
                              agg_sh.at[dstv_all.at[b, j]],
                              ssems.at[b]).wait()

    iload_start(0, 0)
    iload_wait(0, 0)
    fold_offset(0)
    gather_start(0)

    def _chunk(c, carry):
      b = lax.rem(c, 3)
      bn = lax.rem(c + 1, 3)

      @pl.when(c >= 2)
      def _():
        scatter_wait(bn)

      @pl.when(c + 1 < NCHUNK)
      def _():
        iload_start(c + 1, bn)

      gather_wait(b)

      @pl.when(c + 1 < NCHUNK)
      def _():
        iload_wait(c + 1, bn)
        fold_offset(bn)
        gather_start(bn)

      scale(b)
      scatter_start(b)
      return carry
    lax.fori_loop(0, NCHUNK, _chunk, 0)

    scatter_wait((NCHUNK - 2) % 3)
    scatter_wait((NCHUNK - 1) % 3)
    plsc.subcore_barrier()
    row_base = (cid * 2 + p) * NPR + t * ROWS_PT
    pltpu.sync_copy(agg_sh.at[pl.ds(t * ROWS_PT, ROWS_PT)],
                    out_r.at[pl.ds(row_base, ROWS_PT)])

  passes = [
      (0, src_r, dst_r, w_r, True, tab_a, agg_a),
      (0, src_r, dst_r, w_r, True, tab_b, agg_b),
      (1, src2_r, dst2_r, w2_r, False, tab_a, agg_a),
      (1, src2_r, dst2_r, w2_r, False, tab_b, agg_b),
  ]
  for pi, args in enumerate(passes):
    _edge_pass(*args)
    if pi < 3:
      _zero_agg()
      plsc.subcore_barrier()


@jax.jit
def _sc_aggregate(src1, dst1, w1, src2, dst2, w2, tab_a, tab_b):
  mesh = plsc.VectorSubcoreMesh(core_axis_name="c", subcore_axis_name="s",
                                num_cores=2, num_subcores=16)
  f32 = jnp.float32
  kern = pl.kernel(
      _sc_body,
      out_type=[
          jax.ShapeDtypeStruct((4 * NPR, HD), f32),
          jax.ShapeDtypeStruct((4 * NPR, HD), f32),
          jax.ShapeDtypeStruct((NP,), f32),
      ],
      mesh=mesh,
      compiler_params=pltpu.CompilerParams(needs_layout_passes=False,
                                           use_tc_tiling_on_sc=False),
      scratch_types=[
          pltpu.VMEM((3, CH, HD), f32),
          pltpu.VMEM((3, NSUB, SB), jnp.int32),
          pltpu.VMEM((3, NSUB, SB), jnp.int32),
          pltpu.VMEM((3, NSUB, SB), f32),
          pltpu.VMEM((NP,), f32),
          pltpu.VMEM((NP,), f32),
          pltpu.VMEM((NP,), f32),
          pltpu.VMEM((BINS_PT,), f32),
          pltpu.VMEM((BINS_PT,), f32),
          pltpu.VMEM((64, HD), f32),
          pltpu.VMEM_SHARED((NPR, HD), f32),
          pltpu.HBM((16 * NP,), f32),
          pltpu.HBM((16 * NP,), f32),
          pltpu.VMEM_SHARED((NP,), f32),
          pltpu.SemaphoreType.DMA((3,)),
          pltpu.SemaphoreType.DMA((3,)),
          pltpu.SemaphoreType.DMA((3,)),
      ],
  )
  return kern(src1, dst1, w1, src2, dst2, w2, tab_a, tab_b)


_R = 1000
_G = N_NODES // _R


def _prelu(z, a):
  return jnp.where(z >= 0, z, a * z)


def _means_body(a1_r, a2_r, deg_r, W1_r, b1_r, aa1_r, W2_r, b2_r, aa2_r,
                s1_r, s2_r):
  i = pl.program_id(0)
  nd = lax.rsqrt(jnp.maximum(deg_r[...], 1.0))
  z1 = jnp.dot(a1_r[...] * nd, W1_r[...],
               preferred_element_type=jnp.float32) + b1_r[...]
  h1 = _prelu(z1, aa1_r[0, 0])
  z2 = jnp.dot(a2_r[...], W2_r[...],
               preferred_element_type=jnp.float32) + b2_r[...]
  h2 = _prelu(z2, aa2_r[0, 0])

  @pl.when(i == 0)
  def _():
    s1_r[...] = jnp.zeros_like(s1_r)
    s2_r[...] = jnp.zeros_like(s2_r)

  s1_r[...] += jnp.sum(h1, axis=0, keepdims=True)
  s2_r[...] += jnp.sum(h2, axis=0, keepdims=True)


@jax.jit
def _tc_means(agg1, agg2, deg, W1, b1, a1, W2, b2, a2):
  f32 = jnp.float32
  blk = lambda r, c: pl.BlockSpec((r, c), lambda i: (i, 0))
  full = lambda r, c: pl.BlockSpec((r, c), lambda i: (0, 0))
  return pl.pallas_call(
      _means_body,
      grid=(_G,),
      in_specs=[blk(_R, IN_DIM), blk(_R, IN_DIM), blk(_R, 1),
                full(IN_DIM, OUT_DIM), full(1, OUT_DIM), full(1, 1),
                full(IN_DIM, OUT_DIM), full(1, OUT_DIM), full(1, 1)],
      out_specs=[full(1, OUT_DIM), full(1, OUT_DIM)],
      out_shape=[jax.ShapeDtypeStruct((1, OUT_DIM), f32),
                 jax.ShapeDtypeStruct((1, OUT_DIM), f32)],
  )(agg1, agg2, deg, W1, b1, a1, W2, b2, a2)


def _bilin_body(Wb_r, c_r, v_r):
  v_r[...] = jnp.dot(Wb_r[...], c_r[...], preferred_element_type=jnp.float32)


@jax.jit
def _tc_bilin_vec(Wb, c12):
  return pl.pallas_call(
      _bilin_body,
      out_shape=jax.ShapeDtypeStruct((OUT_DIM, 2), jnp.float32),
  )(Wb, c12)


def _scores_body(a1_r, a2_r, a3_r, a4_r, deg_r, W1_r, b1_r, aa1_r,
                 W2_r, b2_r, aa2_r, v_r, bb_r,
                 o1_r, o2_r, o3_r, o4_r):
  nd = lax.rsqrt(jnp.maximum(deg_r[...], 1.0))
  dotf = lambda x, w: jnp.dot(x, w, preferred_element_type=jnp.float32)
  h1 = _prelu(dotf(a1_r[...] * nd, W1_r[...]) + b1_r[...], aa1_r[0, 0])
  h2 = _prelu(dotf(a2_r[...], W2_r[...]) + b2_r[...], aa2_r[0, 0])
  h3 = _prelu(dotf(a3_r[...] * nd, W1_r[...]) + b1_r[...], aa1_r[0, 0])
  h4 = _prelu(dotf(a4_r[...], W2_r[...]) + b2_r[...], aa2_r[0, 0])
  v1 = v_r[:, 0:1]
  v2 = v_r[:, 1:2]
  bb = bb_r[0, 0]
  o1_r[...] = dotf(h2, v1) + bb
  o2_r[...] = dotf(h1, v2) + bb
  o3_r[...] = dotf(h4, v1) + bb
  o4_r[...] = dotf(h3, v2) + bb


@jax.jit
def _tc_scores(agg1, agg2, agg3, agg4, deg, W1, b1, a1, W2, b2, a2, v12, bb):
  f32 = jnp.float32
  blk = lambda r, c: pl.BlockSpec((r, c), lambda i: (i, 0))
  full = lambda r, c: pl.BlockSpec((r, c), lambda i: (0, 0))
  outs = pl.pallas_call(
      _scores_body,
      grid=(_G,),
      in_specs=[blk(_R, IN_DIM)] * 4 + [blk(_R, 1),
                full(IN_DIM, OUT_DIM), full(1, OUT_DIM), full(1, 1),
                full(IN_DIM, OUT_DIM), full(1, OUT_DIM), full(1, 1),
                full(OUT_DIM, 2), full(1, 1)],
      out_specs=[blk(_R, 1)] * 4,
      out_shape=[jax.ShapeDtypeStruct((N_NODES, 1), f32)] * 4,
  )(agg1, agg2, agg3, agg4, deg, W1, b1, a1, W2, b2, a2, v12, bb)
  return outs


def kernel(feat, shuf_feat, edge_index_1, edge_index_2, edge_weight_1,
           edge_weight_2, W1, b1, a1, W2, b2, a2, Wb, bb):
  f32 = jnp.float32
  esh = (16, NCHUNK, NSUB, SB)
  src1 = edge_index_1[0].astype(jnp.int32).reshape(esh)
  dst1 = edge_index_1[1].astype(jnp.int32).reshape(esh)
  src2 = edge_index_2[0].astype(jnp.int32).reshape(esh)
  dst2 = edge_index_2[1].astype(jnp.int32).reshape(esh)
  w1 = edge_weight_1.reshape(esh)
  w2 = edge_weight_2.reshape(esh)
  tab = jnp.concatenate([feat, shuf_feat], axis=0)
  tab_a = tab[:, :HD]
  tab_b = tab[:, HD:]

  agg_ha, agg_hb, deg = _sc_aggregate(src1, dst1, w1, src2, dst2, w2,
                                      tab_a, tab_b)
  aggs = jnp.concatenate([agg_ha, agg_hb], axis=1)
  agg1 = aggs[0 * NPR:0 * NPR + N_NODES]
  agg2 = aggs[1 * NPR:1 * NPR + N_NODES]
  agg3 = aggs[2 * NPR:2 * NPR + N_NODES]
  agg4 = aggs[3 * NPR:3 * NPR + N_NODES]
  degc = deg[:N_NODES].reshape(N_NODES, 1)

  b1r = b1.reshape(1, OUT_DIM)
  b2r = b2.reshape(1, OUT_DIM)
  a1r = a1.reshape(1, 1)
  a2r = a2.reshape(1, 1)
  bbr = bb.reshape(1, 1)

  s1, s2 = _tc_means(agg1, agg2, degc, W1, b1r, a1r, W2, b2r, a2r)
  c1 = jax.nn.sigmoid(s1[0] / N_NODES)
  c2 = jax.nn.sigmoid(s2[0] / N_NODES)
  c12 = jnp.stack([c1, c2], axis=1)
  v12 = _tc_bilin_vec(Wb, c12)

  o1, o2, o3, o4 = _tc_scores(agg1, agg2, agg3, agg4, degc,
                              W1, b1r, a1r, W2, b2r, a2r, v12, bbr)
  return jnp.concatenate([o1[:, 0], o2[:, 0], o3[:, 0], o4[:, 0]])
